# baseline (speedup 1.0000x reference)
import functools

import jax
import jax.numpy as jnp
from jax import lax
from jax.experimental import pallas as pl
from jax.experimental.pallas import tpu as pltpu
from jax.experimental.pallas import tpu_sc as plsc

B = 4096
L = 200
D = 768
NC = 2
NS = 16
NW = NC * NS
ROWS_PER_W = B // NW
G = 40
CHUNKS_PER_ROW = L // G
NCHUNK = ROWS_PER_W * CHUNKS_PER_ROW
NLANE = 16
ND = D // NLANE
INV_L = 1.0 / L


def _sc_body(ids_hbm, table_hbm, out_hbm, idx_v, buf0, buf1, acc_v, sem0, sem1):
    wid = lax.axis_index("s") * NC + lax.axis_index("c")

    pltpu.sync_copy(ids_hbm.at[wid], idx_v)

    def _start(g, buf, sem):
        pltpu.make_async_copy(
            table_hbm.at[idx_v.at[pl.ds(g * G, G)]], buf, sem).start()

    def _wait(buf, sem):
        pltpu.make_async_copy(
            table_hbm.at[idx_v.at[pl.ds(0, G)]], buf, sem).wait()

    def _zero_acc():
        def z(d, _):
            acc_v[pl.ds(d * NLANE, NLANE)] = jnp.zeros((NLANE,), jnp.float32)
            return 0
        lax.fori_loop(0, ND, z, 0)

    def _accum(buf):
        def d_body(du, _):
            for j in range(4):
                sl = pl.ds((du * 4 + j) * NLANE, NLANE)
                a0 = buf[0, sl] + buf[1, sl]
                a1 = buf[2, sl] + buf[3, sl]
                for k in range(4, G, 2):
                    a0 = a0 + buf[k, sl]
                    a1 = a1 + buf[k + 1, sl]
                plsc.addupdate(acc_v.at[sl], a0 + a1)
            return 0
        lax.fori_loop(0, ND // 4, d_body, 0)

    def _maybe_finish(cnt, row):
        @pl.when(cnt == CHUNKS_PER_ROW - 1)
        def _():
            def s(d, _):
                sl = pl.ds(d * NLANE, NLANE)
                acc_v[sl] = acc_v[sl] * jnp.float32(INV_L)
                return 0
            lax.fori_loop(0, ND, s, 0)
            pltpu.sync_copy(acc_v, out_hbm.at[pl.ds(row * D, D)])
            _zero_acc()
        done = cnt == CHUNKS_PER_ROW - 1
        return jnp.where(done, 0, cnt + 1), jnp.where(done, row + 1, row)

    _zero_acc()
    _start(0, buf0, sem0)

    def pair(gp, carry):
        cnt, row = carry
        g0 = 2 * gp
        _start(g0 + 1, buf1, sem1)
        _wait(buf0, sem0)
        _accum(buf0)
        cnt, row = _maybe_finish(cnt, row)

        @pl.when(g0 + 2 < NCHUNK)
        def _():
            _start(g0 + 2, buf0, sem0)
        _wait(buf1, sem1)
        _accum(buf1)
        cnt, row = _maybe_finish(cnt, row)
        return cnt, row

    lax.fori_loop(0, NCHUNK // 2, pair,
                  (jnp.int32(0), (wid * ROWS_PER_W).astype(jnp.int32)))


@jax.jit
def kernel(ids, table):
    ids3 = ids.reshape(NW, NCHUNK * G)
    mesh = plsc.VectorSubcoreMesh(core_axis_name="c", subcore_axis_name="s")
    out = pl.kernel(
        _sc_body,
        out_type=jax.ShapeDtypeStruct((B * D,), jnp.float32),
        mesh=mesh,
        scratch_types=[
            pltpu.VMEM((NCHUNK * G,), jnp.int32),
            pltpu.VMEM((G, D), jnp.float32),
            pltpu.VMEM((G, D), jnp.float32),
            pltpu.VMEM((D,), jnp.float32),
            pltpu.SemaphoreType.DMA,
            pltpu.SemaphoreType.DMA,
        ],
    )(ids3, table)
    return out.reshape(B, D)


# speedup vs baseline: 1.2142x; 1.2142x over previous
import functools

import jax
import jax.numpy as jnp
from jax import lax
from jax.experimental import pallas as pl
from jax.experimental.pallas import tpu as pltpu
from jax.experimental.pallas import tpu_sc as plsc

B = 4096
L = 200
D = 768
NC = 2
NS = 16
NW = NC * NS
ROWS_PER_W = B // NW
G = 40
CHUNKS_PER_ROW = L // G
NCHUNK = ROWS_PER_W * CHUNKS_PER_ROW
NLANE = 16
ND = D // NLANE
INV_L = 1.0 / L


def _sc_body(ids_hbm, table_hbm, out_hbm, idx_v, buf0, buf1, acc_v, sem0, sem1):
    wid = lax.axis_index("s") * NC + lax.axis_index("c")

    pltpu.sync_copy(ids_hbm.at[wid], idx_v)

    def _start(g, buf, sem):
        pltpu.make_async_copy(
            table_hbm.at[idx_v.at[pl.ds(g * G, G)]], buf, sem).start()

    def _wait(buf, sem):
        pltpu.make_async_copy(
            table_hbm.at[idx_v.at[pl.ds(0, G)]], buf, sem).wait()

    def _zero_acc():
        def z(d, _):
            acc_v[pl.ds(d * NLANE, NLANE)] = jnp.zeros((NLANE,), jnp.float32)
            return 0
        lax.fori_loop(0, ND, z, 0)

    def _accum(buf):
        def d_body(du, _):
            for j in range(2):
                sl = pl.ds((du * 2 + j) * NLANE, NLANE)
                a = [buf[k, sl] for k in range(4)]
                for k in range(4, G):
                    a[k % 4] = a[k % 4] + buf[k, sl]
                acc_v[sl] = acc_v[sl] + ((a[0] + a[1]) + (a[2] + a[3]))
            return 0
        lax.fori_loop(0, ND // 2, d_body, 0)

    def _maybe_finish(cnt, row):
        @pl.when(cnt == CHUNKS_PER_ROW - 1)
        def _():
            def s(d, _):
                sl = pl.ds(d * NLANE, NLANE)
                acc_v[sl] = acc_v[sl] * jnp.float32(INV_L)
                return 0
            lax.fori_loop(0, ND, s, 0)
            pltpu.sync_copy(acc_v, out_hbm.at[pl.ds(row * D, D)])
            _zero_acc()
        done = cnt == CHUNKS_PER_ROW - 1
        return jnp.where(done, 0, cnt + 1), jnp.where(done, row + 1, row)

    _zero_acc()
    _start(0, buf0, sem0)

    def pair(gp, carry):
        cnt, row = carry
        g0 = 2 * gp
        _start(g0 + 1, buf1, sem1)
        _wait(buf0, sem0)
        _accum(buf0)
        cnt, row = _maybe_finish(cnt, row)

        @pl.when(g0 + 2 < NCHUNK)
        def _():
            _start(g0 + 2, buf0, sem0)
        _wait(buf1, sem1)
        _accum(buf1)
        cnt, row = _maybe_finish(cnt, row)
        return cnt, row

    lax.fori_loop(0, NCHUNK // 2, pair,
                  (jnp.int32(0), (wid * ROWS_PER_W).astype(jnp.int32)))


@jax.jit
def kernel(ids, table):
    ids3 = ids.reshape(NW, NCHUNK * G)
    mesh = plsc.VectorSubcoreMesh(core_axis_name="c", subcore_axis_name="s")
    out = pl.kernel(
        _sc_body,
        out_type=jax.ShapeDtypeStruct((B * D,), jnp.float32),
        mesh=mesh,
        scratch_types=[
            pltpu.VMEM((NCHUNK * G,), jnp.int32),
            pltpu.VMEM((G, D), jnp.float32),
            pltpu.VMEM((G, D), jnp.float32),
            pltpu.VMEM((D,), jnp.float32),
            pltpu.SemaphoreType.DMA,
            pltpu.SemaphoreType.DMA,
        ],
    )(ids3, table)
    return out.reshape(B, D)


# speedup vs baseline: 1.3520x; 1.1135x over previous
import functools

import jax
import jax.numpy as jnp
from jax import lax
from jax.experimental import pallas as pl
from jax.experimental.pallas import tpu as pltpu
from jax.experimental.pallas import tpu_sc as plsc

B = 4096
L = 200
D = 768
NC = 2
NS = 16
NW = NC * NS
ROWS_PER_W = B // NW
G = 40
CHUNKS_PER_ROW = L // G
NCHUNK = ROWS_PER_W * CHUNKS_PER_ROW
NLANE = 16
ND = D // NLANE
INV_L = 1.0 / L


def _sc_body(ids_hbm, table_hbm, out_hbm, idx_v, buf0, buf1, acc_v, sem0, sem1):
    wid = lax.axis_index("s") * NC + lax.axis_index("c")

    pltpu.sync_copy(ids_hbm.at[wid], idx_v)

    def _start(g, buf, sem):
        pltpu.make_async_copy(
            table_hbm.at[idx_v.at[pl.ds(g * G, G)]], buf, sem).start()

    def _wait(buf, sem):
        pltpu.make_async_copy(
            table_hbm.at[idx_v.at[pl.ds(0, G)]], buf, sem).wait()

    def _zero_acc():
        def z(d, _):
            acc_v[pl.ds(d * NLANE, NLANE)] = jnp.zeros((NLANE,), jnp.float32)
            return 0
        lax.fori_loop(0, ND, z, 0)

    def _accum(buf):
        @functools.partial(plsc.parallel_loop, 0, ND, unroll=2)
        def _(d):
            sl = pl.ds(d * NLANE, NLANE)
            a = [buf[k, sl] for k in range(4)]
            for k in range(4, G):
                a[k % 4] = a[k % 4] + buf[k, sl]
            acc_v[sl] = acc_v[sl] + ((a[0] + a[1]) + (a[2] + a[3]))

    def _maybe_finish(cnt, row):
        @pl.when(cnt == CHUNKS_PER_ROW - 1)
        def _():
            def s(d, _):
                sl = pl.ds(d * NLANE, NLANE)
                acc_v[sl] = acc_v[sl] * jnp.float32(INV_L)
                return 0
            lax.fori_loop(0, ND, s, 0)
            pltpu.sync_copy(acc_v, out_hbm.at[pl.ds(row * D, D)])
            _zero_acc()
        done = cnt == CHUNKS_PER_ROW - 1
        return jnp.where(done, 0, cnt + 1), jnp.where(done, row + 1, row)

    _zero_acc()
    _start(0, buf0, sem0)

    def pair(gp, carry):
        cnt, row = carry
        g0 = 2 * gp
        _start(g0 + 1, buf1, sem1)
        _wait(buf0, sem0)
        _accum(buf0)
        cnt, row = _maybe_finish(cnt, row)

        @pl.when(g0 + 2 < NCHUNK)
        def _():
            _start(g0 + 2, buf0, sem0)
        _wait(buf1, sem1)
        _accum(buf1)
        cnt, row = _maybe_finish(cnt, row)
        return cnt, row

    lax.fori_loop(0, NCHUNK // 2, pair,
                  (jnp.int32(0), (wid * ROWS_PER_W).astype(jnp.int32)))


@jax.jit
def kernel(ids, table):
    ids3 = ids.reshape(NW, NCHUNK * G)
    mesh = plsc.VectorSubcoreMesh(core_axis_name="c", subcore_axis_name="s")
    out = pl.kernel(
        _sc_body,
        out_type=jax.ShapeDtypeStruct((B * D,), jnp.float32),
        mesh=mesh,
        scratch_types=[
            pltpu.VMEM((NCHUNK * G,), jnp.int32),
            pltpu.VMEM((G, D), jnp.float32),
            pltpu.VMEM((G, D), jnp.float32),
            pltpu.VMEM((D,), jnp.float32),
            pltpu.SemaphoreType.DMA,
            pltpu.SemaphoreType.DMA,
        ],
    )(ids3, table)
    return out.reshape(B, D)
